# baseline (speedup 1.0000x reference)
import functools

import jax
import jax.numpy as jnp
from jax import lax
from jax.experimental import pallas as pl
from jax.experimental.pallas import tpu as pltpu
from jax.experimental.pallas import tpu_sc as plsc

_SCALE = 8.0


@functools.lru_cache(maxsize=None)
def _make_sc_kernel(N, S, V, D):
    info = plsc.get_sparse_core_info()
    NC, NS, L = info.num_cores, info.num_subcores, info.num_lanes
    NW = NC * NS
    assert N % NW == 0 and D % L == 0 and S % 8 == 0
    rows_per_w = N // NW
    n_steps = rows_per_w
    assert n_steps % 2 == 0 and n_steps >= 4
    ST = S // 8
    pieces = []
    off = 0
    while off < S:
        pieces.append((off, min(128, S - off)))
        off += min(128, S - off)
    mesh = plsc.VectorSubcoreMesh(core_axis_name="c", subcore_axis_name="s")

    @functools.partial(
        pl.kernel,
        mesh=mesh,
        compiler_params=pltpu.CompilerParams(use_tc_tiling_on_sc=False),
        out_type=jax.ShapeDtypeStruct((N, ST, 8, 2 * D), jnp.float32),
        scratch_types=[
            pltpu.VMEM((2, 1, S), jnp.int32),
            pltpu.VMEM((2, S, D), jnp.float32),
            pltpu.VMEM((2, ST, 8, 2 * D), jnp.float32),
            pltpu.SemaphoreType.DMA,
            pltpu.SemaphoreType.DMA,
        ],
    )
    def k(idx_hbm, table_hbm, out_hbm, idx_v, rows_v, tiled_v, gsem, ssem):
        wid = lax.axis_index("s") * NC + lax.axis_index("c")
        base = wid * rows_per_w

        def load_fire(g, b):
            pltpu.sync_copy(idx_hbm.at[pl.ds(base + g, 1)], idx_v.at[b])
            for (o, n) in pieces:
                pltpu.async_copy(
                    table_hbm.at[idx_v.at[b, 0, pl.ds(o, n)]],
                    rows_v.at[b, pl.ds(o, n)], gsem)

        def wait_gather(b):
            for (o, n) in pieces:
                pltpu.make_async_copy(
                    table_hbm.at[idx_v.at[b, 0, pl.ds(o, n)]],
                    rows_v.at[b, pl.ds(o, n)], gsem).wait()

        def fire_store(g, b):
            pltpu.async_copy(tiled_v.at[b], out_hbm.at[base + g], ssem)

        def wait_store(g, b):
            pltpu.make_async_copy(tiled_v.at[b], out_hbm.at[base + g],
                                  ssem).wait()

        def scale(b):
            @pl.loop(0, S, unroll=8)
            def _scale_row(i):
                t = i // 8
                r = i % 8
                for j in range(D // L):
                    sl = pl.ds(j * L, L)
                    tiled_v[b, t, r, sl] = rows_v[b, i, sl] * _SCALE

        load_fire(0, 0)

        @pl.loop(0, n_steps // 2)
        def _pair(g2):
            g = g2 * 2
            wait_gather(0)

            @pl.when(g2 >= 1)
            def _():
                wait_store(g - 1, 1)

            load_fire(g + 1, 1)
            scale(0)
            fire_store(g, 0)
            wait_gather(1)
            wait_store(g, 0)

            @pl.when(g2 + 1 < n_steps // 2)
            def _():
                load_fire(g + 2, 0)

            scale(1)
            fire_store(g + 1, 1)

        wait_store(n_steps - 1, 1)

    return k


def kernel(inputs, table):
    N, S = inputs.shape
    V, D = table.shape
    out4 = _make_sc_kernel(N, S, V, D)(inputs.astype(jnp.int32), table)
    return out4[:, :, :, :D].reshape(N, S, D)
